# baseline (speedup 1.0000x reference)
import functools

import jax
import jax.numpy as jnp
from jax import lax
from jax.experimental import pallas as pl
from jax.experimental.pallas import tpu as pltpu
from jax.experimental.pallas import tpu_sc as plsc

N_N = 10000
N_E = 160000
D = 128
NG = 8
LANES = 16
NW = 32


def _mm3_body(a_ref, w_ref, b_ref, o1_ref, o2_ref, o3_ref):
    r = jnp.dot(a_ref[...], w_ref[...], preferred_element_type=jnp.float32)
    r = r + b_ref[0:1, :]
    o1_ref[...] = r[:, 0 * D:1 * D]
    o2_ref[...] = r[:, 1 * D:2 * D]
    o3_ref[...] = r[:, 2 * D:3 * D]


def _mm3(a, w, b, br):
    R = a.shape[0]
    bt = jnp.broadcast_to(b.reshape(1, 3 * D), (8, 3 * D))
    return pl.pallas_call(
        _mm3_body,
        grid=(R // br,),
        in_specs=[pl.BlockSpec((br, D), lambda i: (i, 0)),
                  pl.BlockSpec((D, 3 * D), lambda i: (0, 0)),
                  pl.BlockSpec((8, 3 * D), lambda i: (0, 0))],
        out_specs=[pl.BlockSpec((br, D), lambda i: (i, 0))] * 3,
        out_shape=[jax.ShapeDtypeStruct((R, D), jnp.float32)] * 3,
    )(a, w, bt)


def _mm1_body(a_ref, w_ref, b_ref, o_ref):
    r = jnp.dot(a_ref[...], w_ref[...], preferred_element_type=jnp.float32)
    o_ref[...] = r + b_ref[0:1, :]


def _mm1(a, w, b, br):
    R = a.shape[0]
    bt = jnp.broadcast_to(b.reshape(1, D), (8, D))
    return pl.pallas_call(
        _mm1_body,
        grid=(R // br,),
        in_specs=[pl.BlockSpec((br, D), lambda i: (i, 0)),
                  pl.BlockSpec((D, D), lambda i: (0, 0)),
                  pl.BlockSpec((8, D), lambda i: (0, 0))],
        out_specs=pl.BlockSpec((br, D), lambda i: (i, 0)),
        out_shape=jax.ShapeDtypeStruct((R, D), jnp.float32),
    )(a, w, bt)


def _outpre_body(sums_ref, su_ref, o_ref, acc_ref):
    i = pl.program_id(0)
    o = sums_ref[1] / (sums_ref[0] + 1e-6) + su_ref[...]
    o_ref[...] = o
    br = o.shape[0]
    ps = o.reshape(br // 8, 8, D).sum(axis=0)
    psq = (o * o).reshape(br // 8, 8, D).sum(axis=0)

    @pl.when(i == 0)
    def _():
        acc_ref[...] = jnp.zeros_like(acc_ref)

    acc_ref[0] = acc_ref[0] + ps
    acc_ref[1] = acc_ref[1] + psq


def _outpre(sums, su, br):
    R = su.shape[0]
    return pl.pallas_call(
        _outpre_body,
        grid=(R // br,),
        in_specs=[pl.BlockSpec((2, br, D), lambda i: (0, i, 0)),
                  pl.BlockSpec((br, D), lambda i: (i, 0))],
        out_specs=[pl.BlockSpec((br, D), lambda i: (i, 0)),
                   pl.BlockSpec((2, 8, D), lambda i: (0, 0, 0))],
        out_shape=[jax.ShapeDtypeStruct((R, D), jnp.float32),
                   jax.ShapeDtypeStruct((2, 8, D), jnp.float32)],
    )(sums, su)


def _stats_body(x_ref, acc_ref):
    i = pl.program_id(0)
    o = x_ref[...]
    br = o.shape[0]
    ps = o.reshape(br // 8, 8, D).sum(axis=0)
    psq = (o * o).reshape(br // 8, 8, D).sum(axis=0)

    @pl.when(i == 0)
    def _():
        acc_ref[...] = jnp.zeros_like(acc_ref)

    acc_ref[0] = acc_ref[0] + ps
    acc_ref[1] = acc_ref[1] + psq


def _stats(x, br):
    R = x.shape[0]
    return pl.pallas_call(
        _stats_body,
        grid=(R // br,),
        in_specs=[pl.BlockSpec((br, D), lambda i: (i, 0))],
        out_specs=pl.BlockSpec((2, 8, D), lambda i: (0, 0, 0)),
        out_shape=jax.ShapeDtypeStruct((2, 8, D), jnp.float32),
    )(x)


def _apply_body(src_ref, res_ref, acc_ref, gb_ref, o_ref, *, count):
    mean = acc_ref[0].sum(axis=0, keepdims=True) * (1.0 / count)
    ex2 = acc_ref[1].sum(axis=0, keepdims=True) * (1.0 / count)
    var = ex2 - mean * mean
    scale = gb_ref[0, 0:1, :] / jnp.sqrt(var + 1e-5)
    xh = (src_ref[...] - mean) * scale + gb_ref[1, 0:1, :]
    o_ref[...] = xh / (1.0 + jnp.exp(-xh)) + res_ref[...]


def _apply(src, res, acc, g, b, count, br):
    R = src.shape[0]
    gb = jnp.stack([jnp.broadcast_to(g.reshape(1, D), (8, D)),
                    jnp.broadcast_to(b.reshape(1, D), (8, D))])
    return pl.pallas_call(
        functools.partial(_apply_body, count=float(count)),
        grid=(R // br,),
        in_specs=[pl.BlockSpec((br, D), lambda i: (i, 0)),
                  pl.BlockSpec((br, D), lambda i: (i, 0)),
                  pl.BlockSpec((2, 8, D), lambda i: (0, 0, 0)),
                  pl.BlockSpec((2, 8, D), lambda i: (0, 0, 0))],
        out_specs=pl.BlockSpec((br, D), lambda i: (i, 0)),
        out_shape=jax.ShapeDtypeStruct((R, D), jnp.float32),
    )(src, res, acc, gb)


@functools.lru_cache(maxsize=None)
def _make_edge_pass(N, E):
    EP = E // NW
    K = 128
    NCH = EP // K
    TAIL = EP - NCH * K
    mesh = plsc.VectorSubcoreMesh(core_axis_name="c", subcore_axis_name="s")

    @functools.partial(
        pl.kernel, mesh=mesh,
        compiler_params=pltpu.CompilerParams(use_tc_tiling_on_sc=False),
        out_type=[jax.ShapeDtypeStruct((E, D), jnp.float32),
                  jax.ShapeDtypeStruct((2, NG, E, LANES), jnp.float32)],
        scratch_types=[pltpu.VMEM((K,), jnp.int32),
                       pltpu.VMEM((K,), jnp.int32),
                       pltpu.VMEM((K, D), jnp.float32),
                       pltpu.VMEM((K, D), jnp.float32),
                       pltpu.VMEM((K, D), jnp.float32),
                       pltpu.VMEM((K, D), jnp.float32),
                       pltpu.VMEM((K, D), jnp.float32),
                       pltpu.VMEM((2, NG, K, LANES), jnp.float32),
                       pltpu.SemaphoreType.DMA],
    )
    def edge_pass(ns, du, ey, src, dst, m_out, smq_out,
                  srcv, dstv, av, bv, cv, dv, mv, sv, sem):
        wid = lax.axis_index("s") * 2 + lax.axis_index("c")
        base = wid * EP
        lane = lax.iota(jnp.int32, LANES)

        def do_chunk(eb, k):
            pltpu.sync_copy(src.at[pl.ds(eb, k)], srcv.at[pl.ds(0, k)])
            pltpu.sync_copy(dst.at[pl.ds(eb, k)], dstv.at[pl.ds(0, k)])
            if k < K:
                nv = (k + LANES - 1) // LANES
                for j in range(nv):
                    sl = pl.ds(j * LANES, LANES)
                    keep = (lane + j * LANES) < k
                    srcv[sl] = jnp.where(keep, srcv[sl], 0)
                    dstv[sl] = jnp.where(keep, dstv[sl], 0)
                zero = jnp.zeros((LANES,), jnp.int32)
                for j in range(nv, K // LANES):
                    sl = pl.ds(j * LANES, LANES)
                    srcv[sl] = zero
                    dstv[sl] = zero
            pltpu.async_copy(ns.at[dstv], av, sem).wait()
            pltpu.async_copy(ns.at[srcv], bv, sem).wait()
            pltpu.async_copy(du.at[srcv], cv, sem).wait()
            pltpu.sync_copy(ey.at[pl.ds(eb, k), :], dv.at[pl.ds(0, k), :])

            def row(r, carry):
                for g in range(NG):
                    sl = pl.ds(g * LANES, LANES)
                    mvec = av[r, sl] + bv[r, sl] + dv[r, sl]
                    sg = 1.0 / (1.0 + jnp.exp(-mvec))
                    mv[r, sl] = mvec
                    sv[0, g, r, :] = sg
                    sv[1, g, r, :] = sg * cv[r, sl]
                return carry

            lax.fori_loop(0, k, row, 0)
            pltpu.sync_copy(mv.at[pl.ds(0, k), :], m_out.at[pl.ds(eb, k), :])
            for t in range(2):
                for g in range(NG):
                    pltpu.sync_copy(sv.at[t, g, pl.ds(0, k), :],
                                    smq_out.at[t, g, pl.ds(eb, k), :])

        def chunk(i, carry):
            do_chunk(pl.multiple_of(base + i * K, 8), K)
            return carry

        lax.fori_loop(0, NCH, chunk, 0)
        if TAIL:
            do_chunk(pl.multiple_of(base + NCH * K, 8), TAIL)

    return edge_pass


@functools.lru_cache(maxsize=None)
def _make_scatter_pass(N, E, n_chunks):
    NT = 16
    EP = E // NT
    K = 128
    NCH = EP // K
    TAIL = EP - NCH * K
    NROW = NCH + (1 if TAIL else 0)
    Nc = N // n_chunks
    AR = Nc + 16
    RPT = AR // NT
    DRP = Nc // NT
    ZR = 512
    mesh = plsc.VectorSubcoreMesh(core_axis_name="c", subcore_axis_name="s")

    @functools.partial(
        pl.kernel, mesh=mesh,
        compiler_params=pltpu.CompilerParams(use_tc_tiling_on_sc=False),
        out_type=jax.ShapeDtypeStruct((2, N, D), jnp.float32),
        scratch_types=[pltpu.VMEM((EP,), jnp.int32),
                       pltpu.VMEM((NROW, K), jnp.int32),
                       pltpu.VMEM((K, LANES), jnp.float32),
                       pltpu.VMEM((ZR, LANES), jnp.float32),
                       pltpu.VMEM_SHARED((AR, LANES), jnp.float32)],
    )
    def scatter_pass(smq, dst, sums, dstv_all, idxm, rows, zb, acc):
        cid = lax.axis_index("c")
        sid = lax.axis_index("s")
        tb = sid * EP
        lane = lax.iota(jnp.int32, LANES)

        def zrow(r, carry):
            zb[r, :] = jnp.zeros((LANES,), jnp.float32)
            return carry

        lax.fori_loop(0, ZR, zrow, 0)
        pltpu.sync_copy(dst.at[pl.ds(pl.multiple_of(tb, 8), EP)], dstv_all)

        def masked_idx(dvec, base_n):
            if n_chunks == 1:
                return dvec
            inb = (dvec >= base_n) & (dvec < base_n + Nc)
            return jnp.where(inb, dvec - base_n, Nc + (dvec & 15))

        def build_idx(c):
            base_n = c * Nc

            def irow(r, carry):
                for j in range(K // LANES):
                    dvec = dstv_all[pl.ds(r * K + j * LANES, LANES)]
                    idxm[r, pl.ds(j * LANES, LANES)] = masked_idx(dvec, base_n)
                return carry

            lax.fori_loop(0, NCH, irow, 0)
            if TAIL:
                nv_pad = (K - TAIL) // LANES
                for j in range(K // LANES):
                    sl = pl.ds(j * LANES, LANES)
                    if j < nv_pad:
                        idxm[NCH, sl] = Nc + (lane & 15)
                    else:
                        dvec = dstv_all[pl.ds(EP - K + j * LANES, LANES)]
                        idxm[NCH, sl] = masked_idx(dvec, base_n)

        def zero_acc():
            r0 = sid * RPT
            nfull, rem = divmod(RPT, ZR)
            for jj in range(nfull):
                pltpu.sync_copy(zb, acc.at[pl.ds(r0 + jj * ZR, ZR)])
            if rem:
                pltpu.sync_copy(zb.at[pl.ds(0, rem)],
                                acc.at[pl.ds(r0 + nfull * ZR, rem)])

        def accum(t, g):
            def ch(i, carry):
                eb = pl.multiple_of(tb + i * K, 8)
                pltpu.sync_copy(smq.at[t, g, pl.ds(eb, K), :], rows)
                pltpu.sync_copy(rows, acc.at[idxm.at[i]], add=True)
                return carry

            lax.fori_loop(0, NCH, ch, 0)
            if TAIL:
                lb = pl.multiple_of(tb + EP - K, 8)
                pltpu.sync_copy(smq.at[t, g, pl.ds(lb, K), :], rows)
                pltpu.sync_copy(rows, acc.at[idxm.at[NCH]], add=True)

        def drain(t, c, g):
            n0 = c * Nc + sid * DRP
            pltpu.sync_copy(acc.at[pl.ds(sid * DRP, DRP)],
                            sums.at[t, pl.ds(n0, DRP),
                                    pl.ds(g * LANES, LANES)])

        def run(t):
            for c in range(n_chunks):
                build_idx(c)
                for g in range(NG):
                    zero_acc()
                    plsc.subcore_barrier()
                    accum(t, g)
                    plsc.subcore_barrier()
                    drain(t, c, g)
                    plsc.subcore_barrier()

        @pl.when(cid == 0)
        def _():
            run(0)

        @pl.when(cid == 1)
        def _():
            run(1)

    return scatter_pass


def _egc_layer(x, edge_attr, src, dst, p, n_nodes, n_chunks, br_n, br_e):
    (sg_w, sg_b, eg_w, eg_b, su_w, su_b, du_w, du_b,
     bnn_g, bnn_b, bne_g, bne_b) = p
    E = edge_attr.shape[0]
    w3 = jnp.concatenate([sg_w.T, du_w.T, su_w.T], axis=1)
    b3 = jnp.concatenate([sg_b, du_b, su_b])
    ns, du, su = _mm3(x, w3, b3, br_n)
    ey = _mm1(edge_attr, eg_w.T, eg_b, br_e)
    m, smq = _make_edge_pass(n_nodes, E)(ns, du, ey, src, dst)
    sums = _make_scatter_pass(n_nodes, E, n_chunks)(smq, dst)
    out_pre, nstat = _outpre(sums, su, br_n)
    estat = _stats(m, br_e)
    out = _apply(out_pre, x, nstat, bnn_g, bnn_b, n_nodes, br_n)
    m2 = _apply(m, edge_attr, estat, bne_g, bne_b, E, br_e)
    return out, m2


def kernel(x, y, z, edge_index, lg_edge_index,
           n_sg_w, n_sg_b, n_eg_w, n_eg_b, n_su_w, n_su_b, n_du_w, n_du_b,
           n_bnn_g, n_bnn_b, n_bne_g, n_bne_b,
           e_sg_w, e_sg_b, e_eg_w, e_eg_b, e_su_w, e_su_b, e_du_w, e_du_b,
           e_bnn_g, e_bnn_b, e_bne_g, e_bne_b):
    src1, dst1 = edge_index[0], edge_index[1]
    src2, dst2 = lg_edge_index[0], lg_edge_index[1]
    pn = (n_sg_w, n_sg_b, n_eg_w, n_eg_b, n_su_w, n_su_b, n_du_w, n_du_b,
          n_bnn_g, n_bnn_b, n_bne_g, n_bne_b)
    pe = (e_sg_w, e_sg_b, e_eg_w, e_eg_b, e_su_w, e_su_b, e_du_w, e_du_b,
          e_bnn_g, e_bnn_b, e_bne_g, e_bne_b)
    x1, m2 = _egc_layer(x, y, src1, dst1, pn, N_N, 1, br_n=2000, br_e=2000)
    y1, z1 = _egc_layer(m2, z, src2, dst2, pe, N_E, 2, br_n=2000, br_e=2000)
    return (x1, y1, z1)


# speedup vs baseline: 1.3841x; 1.3841x over previous
import functools

import jax
import jax.numpy as jnp
from jax import lax
from jax.experimental import pallas as pl
from jax.experimental.pallas import tpu as pltpu
from jax.experimental.pallas import tpu_sc as plsc

N_N = 10000
N_E = 160000
D = 128
NG = 8
LANES = 16
NW = 32


def _mm3_body(a_ref, w_ref, b_ref, o1_ref, o2_ref, o3_ref):
    r = jnp.dot(a_ref[...], w_ref[...], preferred_element_type=jnp.float32)
    r = r + b_ref[0:1, :]
    o1_ref[...] = r[:, 0 * D:1 * D]
    o2_ref[...] = r[:, 1 * D:2 * D]
    o3_ref[...] = r[:, 2 * D:3 * D]


def _mm3(a, w, b, br):
    R = a.shape[0]
    bt = jnp.broadcast_to(b.reshape(1, 3 * D), (8, 3 * D))
    return pl.pallas_call(
        _mm3_body,
        grid=(R // br,),
        in_specs=[pl.BlockSpec((br, D), lambda i: (i, 0)),
                  pl.BlockSpec((D, 3 * D), lambda i: (0, 0)),
                  pl.BlockSpec((8, 3 * D), lambda i: (0, 0))],
        out_specs=[pl.BlockSpec((br, D), lambda i: (i, 0))] * 3,
        out_shape=[jax.ShapeDtypeStruct((R, D), jnp.float32)] * 3,
    )(a, w, bt)


def _mm1_body(a_ref, w_ref, b_ref, o_ref):
    r = jnp.dot(a_ref[...], w_ref[...], preferred_element_type=jnp.float32)
    o_ref[...] = r + b_ref[0:1, :]


def _mm1(a, w, b, br):
    R = a.shape[0]
    bt = jnp.broadcast_to(b.reshape(1, D), (8, D))
    return pl.pallas_call(
        _mm1_body,
        grid=(R // br,),
        in_specs=[pl.BlockSpec((br, D), lambda i: (i, 0)),
                  pl.BlockSpec((D, D), lambda i: (0, 0)),
                  pl.BlockSpec((8, D), lambda i: (0, 0))],
        out_specs=pl.BlockSpec((br, D), lambda i: (i, 0)),
        out_shape=jax.ShapeDtypeStruct((R, D), jnp.float32),
    )(a, w, bt)


def _outpre_body(sums_ref, su_ref, o_ref, acc_ref):
    i = pl.program_id(0)
    o = sums_ref[1] / (sums_ref[0] + 1e-6) + su_ref[...]
    o_ref[...] = o
    br = o.shape[0]
    ps = o.reshape(br // 8, 8, D).sum(axis=0)
    psq = (o * o).reshape(br // 8, 8, D).sum(axis=0)

    @pl.when(i == 0)
    def _():
        acc_ref[...] = jnp.zeros_like(acc_ref)

    acc_ref[0] = acc_ref[0] + ps
    acc_ref[1] = acc_ref[1] + psq


def _outpre(sums, su, br):
    R = su.shape[0]
    return pl.pallas_call(
        _outpre_body,
        grid=(R // br,),
        in_specs=[pl.BlockSpec((2, br, D), lambda i: (0, i, 0)),
                  pl.BlockSpec((br, D), lambda i: (i, 0))],
        out_specs=[pl.BlockSpec((br, D), lambda i: (i, 0)),
                   pl.BlockSpec((2, 8, D), lambda i: (0, 0, 0))],
        out_shape=[jax.ShapeDtypeStruct((R, D), jnp.float32),
                   jax.ShapeDtypeStruct((2, 8, D), jnp.float32)],
    )(sums, su)


def _stats_body(x_ref, acc_ref):
    i = pl.program_id(0)
    o = x_ref[...]
    br = o.shape[0]
    ps = o.reshape(br // 8, 8, D).sum(axis=0)
    psq = (o * o).reshape(br // 8, 8, D).sum(axis=0)

    @pl.when(i == 0)
    def _():
        acc_ref[...] = jnp.zeros_like(acc_ref)

    acc_ref[0] = acc_ref[0] + ps
    acc_ref[1] = acc_ref[1] + psq


def _stats(x, br):
    R = x.shape[0]
    return pl.pallas_call(
        _stats_body,
        grid=(R // br,),
        in_specs=[pl.BlockSpec((br, D), lambda i: (i, 0))],
        out_specs=pl.BlockSpec((2, 8, D), lambda i: (0, 0, 0)),
        out_shape=jax.ShapeDtypeStruct((2, 8, D), jnp.float32),
    )(x)


def _apply_body(src_ref, res_ref, acc_ref, gb_ref, o_ref, *, count):
    mean = acc_ref[0].sum(axis=0, keepdims=True) * (1.0 / count)
    ex2 = acc_ref[1].sum(axis=0, keepdims=True) * (1.0 / count)
    var = ex2 - mean * mean
    scale = gb_ref[0, 0:1, :] / jnp.sqrt(var + 1e-5)
    xh = (src_ref[...] - mean) * scale + gb_ref[1, 0:1, :]
    o_ref[...] = xh / (1.0 + jnp.exp(-xh)) + res_ref[...]


def _apply(src, res, acc, g, b, count, br):
    R = src.shape[0]
    gb = jnp.stack([jnp.broadcast_to(g.reshape(1, D), (8, D)),
                    jnp.broadcast_to(b.reshape(1, D), (8, D))])
    return pl.pallas_call(
        functools.partial(_apply_body, count=float(count)),
        grid=(R // br,),
        in_specs=[pl.BlockSpec((br, D), lambda i: (i, 0)),
                  pl.BlockSpec((br, D), lambda i: (i, 0)),
                  pl.BlockSpec((2, 8, D), lambda i: (0, 0, 0)),
                  pl.BlockSpec((2, 8, D), lambda i: (0, 0, 0))],
        out_specs=pl.BlockSpec((br, D), lambda i: (i, 0)),
        out_shape=jax.ShapeDtypeStruct((R, D), jnp.float32),
    )(src, res, acc, gb)


@functools.lru_cache(maxsize=None)
def _make_edge_pass(N, E):
    EP = E // NW
    K = 64
    F = EP // K
    TAIL = EP - F * K
    EPP = (F + 1) * K
    assert TAIL and TAIL % 8 == 0 and F >= 4
    P = (F - 2) // 2
    mesh = plsc.VectorSubcoreMesh(core_axis_name="c", subcore_axis_name="s")

    @functools.partial(
        pl.kernel, mesh=mesh,
        compiler_params=pltpu.CompilerParams(use_tc_tiling_on_sc=False),
        out_type=jax.ShapeDtypeStruct((E, 3 * D), jnp.float32),
        scratch_types=[pltpu.VMEM((EPP,), jnp.int32),
                       pltpu.VMEM((EPP,), jnp.int32),
                       pltpu.VMEM((2, K, D), jnp.float32),
                       pltpu.VMEM((2, K, D), jnp.float32),
                       pltpu.VMEM((2, K, D), jnp.float32),
                       pltpu.VMEM((2, K, D), jnp.float32),
                       pltpu.VMEM((2, K, 3 * D), jnp.float32),
                       pltpu.SemaphoreType.DMA,
                       pltpu.SemaphoreType.DMA,
                       pltpu.SemaphoreType.DMA,
                       pltpu.SemaphoreType.DMA],
    )
    def edge_pass(ns, du, ey, src, dst, eall,
                  sall, dall, av, bv, cv, dv, ov, si0, si1, so0, so1):
        wid = lax.axis_index("s") * 2 + lax.axis_index("c")
        base = wid * EP
        si = (si0, si1)
        so = (so0, so1)
        lane = lax.iota(jnp.int32, LANES)

        bas8 = pl.multiple_of(base, 8)
        pltpu.sync_copy(src.at[pl.ds(bas8, EP)], sall.at[pl.ds(0, EP)])
        pltpu.sync_copy(dst.at[pl.ds(bas8, EP)], dall.at[pl.ds(0, EP)])
        pv = EP // LANES
        rem = EP - pv * LANES
        if rem:
            sl = pl.ds(pv * LANES, LANES)
            keep = lane < rem
            sall[sl] = jnp.where(keep, sall[sl], 0)
            dall[sl] = jnp.where(keep, dall[sl], 0)
        zero = jnp.zeros((LANES,), jnp.int32)
        for j in range(pv + (1 if rem else 0), EPP // LANES):
            sall[pl.ds(j * LANES, LANES)] = zero
            dall[pl.ds(j * LANES, LANES)] = zero

        def in_copies(i, b, k_ey):
            eb = pl.multiple_of(base + i * K, 8)
            ebl = pl.multiple_of(i * K, 8)
            return [(ns.at[dall.at[pl.ds(ebl, K)]], av.at[b]),
                    (ns.at[sall.at[pl.ds(ebl, K)]], bv.at[b]),
                    (du.at[sall.at[pl.ds(ebl, K)]], cv.at[b]),
                    (ey.at[pl.ds(eb, k_ey), :], dv.at[b, pl.ds(0, k_ey), :])]

        def issue_in(i, b, k_ey=K):
            for s, d_ in in_copies(i, b, k_ey):
                pltpu.async_copy(s, d_, si[b])

        def wait_in(i, b, k_ey=K):
            for s, d_ in in_copies(i, b, k_ey):
                pltpu.make_async_copy(s, d_, si[b]).wait()

        def compute(b):
            def row(r, carry):
                for g in range(NG):
                    sl = pl.ds(g * LANES, LANES)
                    mvec = av[b, r, sl] + bv[b, r, sl] + dv[b, r, sl]
                    sg = 1.0 / (1.0 + jnp.exp(-mvec))
                    ov[b, r, sl] = mvec
                    ov[b, r, pl.ds(D + g * LANES, LANES)] = sg
                    ov[b, r, pl.ds(2 * D + g * LANES, LANES)] = sg * cv[b, r, sl]
                return carry

            lax.fori_loop(0, K, row, 0)

        def issue_out(i, b, k=K):
            eb = pl.multiple_of(base + i * K, 8)
            pltpu.async_copy(ov.at[b, pl.ds(0, k), :],
                             eall.at[pl.ds(eb, k), :], so[b])

        def wait_out(b, k=K):
            pltpu.make_async_copy(ov.at[b, pl.ds(0, k), :],
                                  eall.at[pl.ds(0, k), :], so[b]).wait()

        issue_in(F, 0, TAIL)
        wait_in(F, 0, TAIL)
        compute(0)
        issue_out(F, 0, TAIL)
        wait_out(0, TAIL)

        issue_in(0, 0)
        issue_in(1, 1)

        def pair(j, carry):
            i0 = j * 2
            wait_in(i0, 0)
            compute(0)

            @pl.when(j >= 1)
            def _():
                wait_out(0)

            issue_out(i0, 0)
            issue_in(i0 + 2, 0)
            i1 = i0 + 1
            wait_in(i1, 1)
            compute(1)

            @pl.when(j >= 1)
            def _():
                wait_out(1)

            issue_out(i1, 1)
            issue_in(i1 + 2, 1)
            return carry

        lax.fori_loop(0, P, pair, 0)
        for i in range(2 * P + 2, F):
            issue_in(i, i & 1)
        for i in range(2 * P, F):
            b = i & 1
            wait_in(i, b)
            compute(b)
            wait_out(b)
            issue_out(i, b)
        wait_out(0)
        wait_out(1)

    return edge_pass


@functools.lru_cache(maxsize=None)
def _make_scatter_pass(N, E, n_chunks):
    NT = 16
    EP = E // NT
    K = 128
    F = EP // K
    TAIL = EP - F * K
    NROW = F + 1
    SLK = 3
    SL = SLK * K
    NS = F // SLK
    Nc = N // n_chunks
    AR = Nc + 16
    RPT = AR // NT
    DRP = Nc // NT
    ZR = 256
    assert TAIL == 16 and F % SLK == 0
    assert RPT * NT == AR and DRP * NT == Nc
    mesh = plsc.VectorSubcoreMesh(core_axis_name="c", subcore_axis_name="s")

    @functools.partial(
        pl.kernel, mesh=mesh,
        compiler_params=pltpu.CompilerParams(use_tc_tiling_on_sc=False),
        out_type=jax.ShapeDtypeStruct((2, N, D), jnp.float32),
        scratch_types=[pltpu.VMEM((EP,), jnp.int32),
                       pltpu.VMEM((NROW, K), jnp.int32),
                       pltpu.VMEM((2, SL, LANES), jnp.float32),
                       pltpu.VMEM((ZR, LANES), jnp.float32),
                       pltpu.VMEM_SHARED((AR, LANES), jnp.float32),
                       pltpu.SemaphoreType.DMA,
                       pltpu.SemaphoreType.DMA],
    )
    def scatter_pass(smq, dst, sums, dstv_all, idxm, slab, zb, acc, sb0, sb1):
        cid = lax.axis_index("c")
        sid = lax.axis_index("s")
        tb = sid * EP
        sb = (sb0, sb1)
        lane = lax.iota(jnp.int32, LANES)

        def zrow(r, carry):
            zb[r, :] = jnp.zeros((LANES,), jnp.float32)
            return carry

        lax.fori_loop(0, ZR, zrow, 0)
        pltpu.sync_copy(dst.at[pl.ds(pl.multiple_of(tb, 8), EP)], dstv_all)

        def masked_idx(dvec, base_n):
            if n_chunks == 1:
                return dvec
            inb = (dvec >= base_n) & (dvec < base_n + Nc)
            return jnp.where(inb, dvec - base_n, Nc + (dvec & 15))

        def build_idx(c):
            base_n = c * Nc

            def irow(r, carry):
                for j in range(K // LANES):
                    dvec = dstv_all[pl.ds(r * K + j * LANES, LANES)]
                    idxm[r, pl.ds(j * LANES, LANES)] = masked_idx(dvec, base_n)
                return carry

            lax.fori_loop(0, F, irow, 0)
            nv_pad = (K - TAIL) // LANES
            for j in range(K // LANES):
                sl = pl.ds(j * LANES, LANES)
                if j < nv_pad:
                    idxm[F, sl] = Nc + (lane & 15)
                else:
                    dvec = dstv_all[pl.ds(EP - K + j * LANES, LANES)]
                    idxm[F, sl] = masked_idx(dvec, base_n)

        def zero_acc():
            r0 = sid * RPT
            nfull, rem = divmod(RPT, ZR)

            def zc(jj, carry):
                pltpu.sync_copy(zb, acc.at[pl.ds(r0 + jj * ZR, ZR)])
                return carry

            lax.fori_loop(0, nfull, zc, 0)
            if rem:
                pltpu.sync_copy(zb.at[pl.ds(0, rem)],
                                acc.at[pl.ds(r0 + nfull * ZR, rem)])

        def accum(t, g):
            co = D * (1 + t) + g * LANES

            def slab_copy(i, b):
                return (smq.at[pl.ds(pl.multiple_of(tb + i * SL, 8), SL),
                               pl.ds(co, LANES)],
                        slab.at[b, pl.ds(0, SL), :])

            def issue(i, b):
                s_, d_ = slab_copy(i, b)
                pltpu.async_copy(s_, d_, sb[b])

            def wait(i, b):
                s_, d_ = slab_copy(i, b)
                pltpu.make_async_copy(s_, d_, sb[b]).wait()

            def scatters(i, b):
                def sc(jj, carry):
                    pltpu.sync_copy(slab.at[b, pl.ds(jj * K, K), :],
                                    acc.at[idxm.at[i * SLK + jj]], add=True)
                    return carry

                lax.fori_loop(0, SLK, sc, 0)

            issue(0, 0)
            issue(1, 1)

            def spair(j, carry):
                i0 = 2 * j
                wait(i0, 0)
                scatters(i0, 0)

                @pl.when(i0 + 2 < NS)
                def _():
                    issue(i0 + 2, 0)

                wait(i0 + 1, 1)
                scatters(i0 + 1, 1)

                @pl.when(i0 + 3 < NS)
                def _():
                    issue(i0 + 3, 1)

                return carry

            lax.fori_loop(0, NS // 2, spair, 0)
            ts = smq.at[pl.ds(pl.multiple_of(tb + EP - K, 8), K),
                        pl.ds(co, LANES)]
            td = slab.at[0, pl.ds(0, K), :]
            pltpu.async_copy(ts, td, sb[0])
            pltpu.make_async_copy(ts, td, sb[0]).wait()
            pltpu.sync_copy(slab.at[0, pl.ds(0, K), :],
                            acc.at[idxm.at[F]], add=True)

        def drain(t, c, g):
            n0 = c * Nc + sid * DRP
            pltpu.sync_copy(acc.at[pl.ds(sid * DRP, DRP)],
                            sums.at[t, pl.ds(n0, DRP),
                                    pl.ds(g * LANES, LANES)])

        def run(t):
            for c in range(n_chunks):
                build_idx(c)
                for g in range(NG):
                    zero_acc()
                    plsc.subcore_barrier()
                    accum(t, g)
                    plsc.subcore_barrier()
                    drain(t, c, g)
                    plsc.subcore_barrier()

        @pl.when(cid == 0)
        def _():
            run(0)

        @pl.when(cid == 1)
        def _():
            run(1)

    return scatter_pass


def _egc_layer(x, edge_attr, src, dst, p, n_nodes, n_chunks, br_n, br_e):
    (sg_w, sg_b, eg_w, eg_b, su_w, su_b, du_w, du_b,
     bnn_g, bnn_b, bne_g, bne_b) = p
    E = edge_attr.shape[0]
    w3 = jnp.concatenate([sg_w.T, du_w.T, su_w.T], axis=1)
    b3 = jnp.concatenate([sg_b, du_b, su_b])
    ns, du, su = _mm3(x, w3, b3, br_n)
    ey = _mm1(edge_attr, eg_w.T, eg_b, br_e)
    eall = _make_edge_pass(n_nodes, E)(ns, du, ey, src, dst)
    sums = _make_scatter_pass(n_nodes, E, n_chunks)(eall, dst)
    out_pre, nstat = _outpre(sums, su, br_n)
    estat = _stats(eall, br_e)
    out = _apply(out_pre, x, nstat, bnn_g, bnn_b, n_nodes, br_n)
    m2 = _apply(eall, edge_attr, estat, bne_g, bne_b, E, br_e)
    return out, m2


def kernel(x, y, z, edge_index, lg_edge_index,
           n_sg_w, n_sg_b, n_eg_w, n_eg_b, n_su_w, n_su_b, n_du_w, n_du_b,
           n_bnn_g, n_bnn_b, n_bne_g, n_bne_b,
           e_sg_w, e_sg_b, e_eg_w, e_eg_b, e_su_w, e_su_b, e_du_w, e_du_b,
           e_bnn_g, e_bnn_b, e_bne_g, e_bne_b):
    src1, dst1 = edge_index[0], edge_index[1]
    src2, dst2 = lg_edge_index[0], lg_edge_index[1]
    pn = (n_sg_w, n_sg_b, n_eg_w, n_eg_b, n_su_w, n_su_b, n_du_w, n_du_b,
          n_bnn_g, n_bnn_b, n_bne_g, n_bne_b)
    pe = (e_sg_w, e_sg_b, e_eg_w, e_eg_b, e_su_w, e_su_b, e_du_w, e_du_b,
          e_bnn_g, e_bnn_b, e_bne_g, e_bne_b)
    x1, m2 = _egc_layer(x, y, src1, dst1, pn, N_N, 1, br_n=2000, br_e=2000)
    y1, z1 = _egc_layer(m2, z, src2, dst2, pe, N_E, 2, br_n=2000, br_e=2000)
    return (x1, y1, z1)


# speedup vs baseline: 1.3923x; 1.0059x over previous
import functools

import jax
import jax.numpy as jnp
from jax import lax
from jax.experimental import pallas as pl
from jax.experimental.pallas import tpu as pltpu
from jax.experimental.pallas import tpu_sc as plsc

N_N = 10000
N_E = 160000
D = 128
NG = 8
LANES = 16
NW = 32


def _mm3_body(a_ref, w_ref, b_ref, o1_ref, o2_ref, o3_ref):
    r = jnp.dot(a_ref[...], w_ref[...], preferred_element_type=jnp.float32)
    r = r + b_ref[0:1, :]
    o1_ref[...] = r[:, 0 * D:1 * D]
    o2_ref[...] = r[:, 0 * D:2 * D]
    o3_ref[...] = r[:, 2 * D:3 * D]


def _mm3(a, w, b, br):
    R = a.shape[0]
    bt = jnp.broadcast_to(b.reshape(1, 3 * D), (8, 3 * D))
    return pl.pallas_call(
        _mm3_body,
        grid=(R // br,),
        in_specs=[pl.BlockSpec((br, D), lambda i: (i, 0)),
                  pl.BlockSpec((D, 3 * D), lambda i: (0, 0)),
                  pl.BlockSpec((8, 3 * D), lambda i: (0, 0))],
        out_specs=[pl.BlockSpec((br, D), lambda i: (i, 0)),
                   pl.BlockSpec((br, 2 * D), lambda i: (i, 0)),
                   pl.BlockSpec((br, D), lambda i: (i, 0))],
        out_shape=[jax.ShapeDtypeStruct((R, D), jnp.float32),
                   jax.ShapeDtypeStruct((R, 2 * D), jnp.float32),
                   jax.ShapeDtypeStruct((R, D), jnp.float32)],
    )(a, w, bt)


def _mm1_body(a_ref, w_ref, b_ref, o_ref):
    r = jnp.dot(a_ref[...], w_ref[...], preferred_element_type=jnp.float32)
    o_ref[...] = r + b_ref[0:1, :]


def _mm1(a, w, b, br):
    R = a.shape[0]
    bt = jnp.broadcast_to(b.reshape(1, D), (8, D))
    return pl.pallas_call(
        _mm1_body,
        grid=(R // br,),
        in_specs=[pl.BlockSpec((br, D), lambda i: (i, 0)),
                  pl.BlockSpec((D, D), lambda i: (0, 0)),
                  pl.BlockSpec((8, D), lambda i: (0, 0))],
        out_specs=pl.BlockSpec((br, D), lambda i: (i, 0)),
        out_shape=jax.ShapeDtypeStruct((R, D), jnp.float32),
    )(a, w, bt)


def _outpre_body(sums_ref, su_ref, o_ref, acc_ref):
    i = pl.program_id(0)
    o = sums_ref[1] / (sums_ref[0] + 1e-6) + su_ref[...]
    o_ref[...] = o
    br = o.shape[0]
    ps = o.reshape(br // 8, 8, D).sum(axis=0)
    psq = (o * o).reshape(br // 8, 8, D).sum(axis=0)

    @pl.when(i == 0)
    def _():
        acc_ref[...] = jnp.zeros_like(acc_ref)

    acc_ref[0] = acc_ref[0] + ps
    acc_ref[1] = acc_ref[1] + psq


def _outpre(sums, su, br):
    R = su.shape[0]
    return pl.pallas_call(
        _outpre_body,
        grid=(R // br,),
        in_specs=[pl.BlockSpec((2, br, D), lambda i: (0, i, 0)),
                  pl.BlockSpec((br, D), lambda i: (i, 0))],
        out_specs=[pl.BlockSpec((br, D), lambda i: (i, 0)),
                   pl.BlockSpec((2, 8, D), lambda i: (0, 0, 0))],
        out_shape=[jax.ShapeDtypeStruct((R, D), jnp.float32),
                   jax.ShapeDtypeStruct((2, 8, D), jnp.float32)],
    )(sums, su)


def _stats_body(x_ref, acc_ref):
    i = pl.program_id(0)
    o = x_ref[...]
    br = o.shape[0]
    ps = o.reshape(br // 8, 8, D).sum(axis=0)
    psq = (o * o).reshape(br // 8, 8, D).sum(axis=0)

    @pl.when(i == 0)
    def _():
        acc_ref[...] = jnp.zeros_like(acc_ref)

    acc_ref[0] = acc_ref[0] + ps
    acc_ref[1] = acc_ref[1] + psq


def _stats(x, br):
    R = x.shape[0]
    return pl.pallas_call(
        _stats_body,
        grid=(R // br,),
        in_specs=[pl.BlockSpec((br, D), lambda i: (i, 0))],
        out_specs=pl.BlockSpec((2, 8, D), lambda i: (0, 0, 0)),
        out_shape=jax.ShapeDtypeStruct((2, 8, D), jnp.float32),
    )(x)


def _apply_body(src_ref, res_ref, acc_ref, gb_ref, o_ref, *, count):
    mean = acc_ref[0].sum(axis=0, keepdims=True) * (1.0 / count)
    ex2 = acc_ref[1].sum(axis=0, keepdims=True) * (1.0 / count)
    var = ex2 - mean * mean
    scale = gb_ref[0, 0:1, :] / jnp.sqrt(var + 1e-5)
    xh = (src_ref[...] - mean) * scale + gb_ref[1, 0:1, :]
    o_ref[...] = xh / (1.0 + jnp.exp(-xh)) + res_ref[...]


def _apply(src, res, acc, g, b, count, br):
    R = src.shape[0]
    gb = jnp.stack([jnp.broadcast_to(g.reshape(1, D), (8, D)),
                    jnp.broadcast_to(b.reshape(1, D), (8, D))])
    return pl.pallas_call(
        functools.partial(_apply_body, count=float(count)),
        grid=(R // br,),
        in_specs=[pl.BlockSpec((br, D), lambda i: (i, 0)),
                  pl.BlockSpec((br, D), lambda i: (i, 0)),
                  pl.BlockSpec((2, 8, D), lambda i: (0, 0, 0)),
                  pl.BlockSpec((2, 8, D), lambda i: (0, 0, 0))],
        out_specs=pl.BlockSpec((br, D), lambda i: (i, 0)),
        out_shape=jax.ShapeDtypeStruct((R, D), jnp.float32),
    )(src, res, acc, gb)


@functools.lru_cache(maxsize=None)
def _make_edge_pass(N, E):
    EP = E // NW
    K = 64
    F = EP // K
    TAIL = EP - F * K
    EPP = (F + 1) * K
    assert TAIL and TAIL % 8 == 0 and F >= 4
    P = (F - 2) // 2
    mesh = plsc.VectorSubcoreMesh(core_axis_name="c", subcore_axis_name="s")

    @functools.partial(
        pl.kernel, mesh=mesh,
        compiler_params=pltpu.CompilerParams(use_tc_tiling_on_sc=False),
        out_type=jax.ShapeDtypeStruct((E, 3 * D), jnp.float32),
        scratch_types=[pltpu.VMEM((EPP,), jnp.int32),
                       pltpu.VMEM((EPP,), jnp.int32),
                       pltpu.VMEM((2, K, D), jnp.float32),
                       pltpu.VMEM((2, K, 2 * D), jnp.float32),
                       pltpu.VMEM((2, K, D), jnp.float32),
                       pltpu.VMEM((2, K, 3 * D), jnp.float32),
                       pltpu.SemaphoreType.DMA,
                       pltpu.SemaphoreType.DMA,
                       pltpu.SemaphoreType.DMA,
                       pltpu.SemaphoreType.DMA],
    )
    def edge_pass(ns, nd, ey, src, dst, eall,
                  sall, dall, av, bc, dv, ov, si0, si1, so0, so1):
        wid = lax.axis_index("s") * 2 + lax.axis_index("c")
        base = wid * EP
        si = (si0, si1)
        so = (so0, so1)
        lane = lax.iota(jnp.int32, LANES)

        bas8 = pl.multiple_of(base, 8)
        pltpu.sync_copy(src.at[pl.ds(bas8, EP)], sall.at[pl.ds(0, EP)])
        pltpu.sync_copy(dst.at[pl.ds(bas8, EP)], dall.at[pl.ds(0, EP)])
        pv = EP // LANES
        rem = EP - pv * LANES
        if rem:
            sl = pl.ds(pv * LANES, LANES)
            keep = lane < rem
            sall[sl] = jnp.where(keep, sall[sl], 0)
            dall[sl] = jnp.where(keep, dall[sl], 0)
        zero = jnp.zeros((LANES,), jnp.int32)
        for j in range(pv + (1 if rem else 0), EPP // LANES):
            sall[pl.ds(j * LANES, LANES)] = zero
            dall[pl.ds(j * LANES, LANES)] = zero

        def in_copies(i, b, k_ey):
            eb = pl.multiple_of(base + i * K, 8)
            ebl = pl.multiple_of(i * K, 8)
            return [(ns.at[dall.at[pl.ds(ebl, K)]], av.at[b]),
                    (nd.at[sall.at[pl.ds(ebl, K)]], bc.at[b]),
                    (ey.at[pl.ds(eb, k_ey), :], dv.at[b, pl.ds(0, k_ey), :])]

        def issue_in(i, b, k_ey=K):
            for s, d_ in in_copies(i, b, k_ey):
                pltpu.async_copy(s, d_, si[b])

        def wait_in(i, b, k_ey=K):
            for s, d_ in in_copies(i, b, k_ey):
                pltpu.make_async_copy(s, d_, si[b]).wait()

        def compute(b):
            def row(r, carry):
                for g in range(NG):
                    sl = pl.ds(g * LANES, LANES)
                    mvec = av[b, r, sl] + bc[b, r, sl] + dv[b, r, sl]
                    sg = 1.0 / (1.0 + jnp.exp(-mvec))
                    ov[b, r, sl] = mvec
                    ov[b, r, pl.ds(D + g * LANES, LANES)] = sg
                    ov[b, r, pl.ds(2 * D + g * LANES, LANES)] = (
                        sg * bc[b, r, pl.ds(D + g * LANES, LANES)])
                return carry

            lax.fori_loop(0, K, row, 0)

        def issue_out(i, b, k=K):
            eb = pl.multiple_of(base + i * K, 8)
            pltpu.async_copy(ov.at[b, pl.ds(0, k), :],
                             eall.at[pl.ds(eb, k), :], so[b])

        def wait_out(b, k=K):
            pltpu.make_async_copy(ov.at[b, pl.ds(0, k), :],
                                  eall.at[pl.ds(0, k), :], so[b]).wait()

        issue_in(F, 0, TAIL)
        wait_in(F, 0, TAIL)
        compute(0)
        issue_out(F, 0, TAIL)
        wait_out(0, TAIL)

        issue_in(0, 0)
        issue_in(1, 1)

        def pair(j, carry):
            i0 = j * 2
            wait_in(i0, 0)
            compute(0)

            @pl.when(j >= 1)
            def _():
                wait_out(0)

            issue_out(i0, 0)
            issue_in(i0 + 2, 0)
            i1 = i0 + 1
            wait_in(i1, 1)
            compute(1)

            @pl.when(j >= 1)
            def _():
                wait_out(1)

            issue_out(i1, 1)
            issue_in(i1 + 2, 1)
            return carry

        lax.fori_loop(0, P, pair, 0)
        for i in range(2 * P + 2, F):
            issue_in(i, i & 1)
        for i in range(2 * P, F):
            b = i & 1
            wait_in(i, b)
            compute(b)
            wait_out(b)
            issue_out(i, b)
        wait_out(0)
        wait_out(1)

    return edge_pass


@functools.lru_cache(maxsize=None)
def _make_scatter_pass(N, E, n_chunks):
    NT = 16
    EP = E // NT
    K = 128
    F = EP // K
    TAIL = EP - F * K
    NROW = F + 1
    SLK = 3
    SL = SLK * K
    NS = F // SLK
    Nc = N // n_chunks
    DUMP = 512
    AR = Nc + DUMP
    RPT = AR // NT
    DRP = Nc // NT
    ZR = 256
    assert TAIL == 16 and F % SLK == 0 and NS % 2 == 0
    assert RPT * NT == AR and DRP * NT == Nc
    mesh = plsc.VectorSubcoreMesh(core_axis_name="c", subcore_axis_name="s")

    @functools.partial(
        pl.kernel, mesh=mesh,
        compiler_params=pltpu.CompilerParams(use_tc_tiling_on_sc=False),
        out_type=jax.ShapeDtypeStruct((2, N, D), jnp.float32),
        scratch_types=[pltpu.VMEM((EP,), jnp.int32),
                       pltpu.VMEM((NROW, K), jnp.int32),
                       pltpu.VMEM((2, SL, LANES), jnp.float32),
                       pltpu.VMEM((ZR, LANES), jnp.float32),
                       pltpu.VMEM_SHARED((AR, LANES), jnp.float32),
                       pltpu.SemaphoreType.DMA,
                       pltpu.SemaphoreType.DMA],
    )
    def scatter_pass(smq, dst, sums, dstv_all, idxm, slab, zb, acc, sb0, sb1):
        cid = lax.axis_index("c")
        sid = lax.axis_index("s")
        tb = sid * EP
        sb = (sb0, sb1)
        lane = lax.iota(jnp.int32, LANES)

        def zrow(r, carry):
            zb[r, :] = jnp.zeros((LANES,), jnp.float32)
            return carry

        lax.fori_loop(0, ZR, zrow, 0)
        pltpu.sync_copy(dst.at[pl.ds(pl.multiple_of(tb, 8), EP)], dstv_all)

        def masked_idx(dvec, base_n):
            if n_chunks == 1:
                return dvec
            inb = (dvec >= base_n) & (dvec < base_n + Nc)
            return jnp.where(inb, dvec - base_n, Nc + (dvec & (DUMP - 1)))

        def build_idx(c):
            base_n = c * Nc

            def irow(r, carry):
                for j in range(K // LANES):
                    dvec = dstv_all[pl.ds(r * K + j * LANES, LANES)]
                    idxm[r, pl.ds(j * LANES, LANES)] = masked_idx(dvec, base_n)
                return carry

            lax.fori_loop(0, F, irow, 0)
            nv_pad = (K - TAIL) // LANES
            for j in range(K // LANES):
                sl = pl.ds(j * LANES, LANES)
                if j < nv_pad:
                    idxm[F, sl] = Nc + ((lane + j * LANES) & (DUMP - 1))
                else:
                    dvec = dstv_all[pl.ds(EP - K + j * LANES, LANES)]
                    idxm[F, sl] = masked_idx(dvec, base_n)

        def zero_acc():
            r0 = sid * RPT
            nfull, rem = divmod(RPT, ZR)

            def zc(jj, carry):
                pltpu.sync_copy(zb, acc.at[pl.ds(r0 + jj * ZR, ZR)])
                return carry

            lax.fori_loop(0, nfull, zc, 0)
            if rem:
                pltpu.sync_copy(zb.at[pl.ds(0, rem)],
                                acc.at[pl.ds(r0 + nfull * ZR, rem)])

        def accum(t, g):
            co = D * (1 + t) + g * LANES

            def slab_copy(i, b):
                return (smq.at[pl.ds(pl.multiple_of(tb + i * SL, 8), SL),
                               pl.ds(co, LANES)],
                        slab.at[b, pl.ds(0, SL), :])

            def issue(i, b):
                s_, d_ = slab_copy(i, b)
                pltpu.async_copy(s_, d_, sb[b])

            def wait(i, b):
                s_, d_ = slab_copy(i, b)
                pltpu.make_async_copy(s_, d_, sb[b]).wait()

            def scatters(i, b):
                def sc(jj, carry):
                    pltpu.sync_copy(slab.at[b, pl.ds(jj * K, K), :],
                                    acc.at[idxm.at[i * SLK + jj]], add=True)
                    return carry

                lax.fori_loop(0, SLK, sc, 0)

            issue(0, 0)
            issue(1, 1)

            def spair(j, carry):
                i0 = 2 * j
                wait(i0, 0)
                scatters(i0, 0)

                @pl.when(i0 + 2 < NS)
                def _():
                    issue(i0 + 2, 0)

                wait(i0 + 1, 1)
                scatters(i0 + 1, 1)

                @pl.when(i0 + 3 < NS)
                def _():
                    issue(i0 + 3, 1)

                return carry

            lax.fori_loop(0, NS // 2, spair, 0)
            ts = smq.at[pl.ds(pl.multiple_of(tb + EP - K, 8), K),
                        pl.ds(co, LANES)]
            td = slab.at[0, pl.ds(0, K), :]
            pltpu.async_copy(ts, td, sb[0])
            pltpu.make_async_copy(ts, td, sb[0]).wait()
            pltpu.sync_copy(slab.at[0, pl.ds(0, K), :],
                            acc.at[idxm.at[F]], add=True)

        def drain(t, c, g):
            n0 = c * Nc + sid * DRP
            pltpu.sync_copy(acc.at[pl.ds(sid * DRP, DRP)],
                            sums.at[t, pl.ds(n0, DRP),
                                    pl.ds(g * LANES, LANES)])

        def run(t):
            for c in range(n_chunks):
                build_idx(c)
                for g in range(NG):
                    zero_acc()
                    plsc.subcore_barrier()
                    accum(t, g)
                    plsc.subcore_barrier()
                    drain(t, c, g)
                    plsc.subcore_barrier()

        @pl.when(cid == 0)
        def _():
            run(0)

        @pl.when(cid == 1)
        def _():
            run(1)

    return scatter_pass


def _egc_layer(x, edge_attr, src, dst, p, n_nodes, n_chunks, br_n, br_e):
    (sg_w, sg_b, eg_w, eg_b, su_w, su_b, du_w, du_b,
     bnn_g, bnn_b, bne_g, bne_b) = p
    E = edge_attr.shape[0]
    w3 = jnp.concatenate([sg_w.T, du_w.T, su_w.T], axis=1)
    b3 = jnp.concatenate([sg_b, du_b, su_b])
    ns, nd, su = _mm3(x, w3, b3, br_n)
    ey = _mm1(edge_attr, eg_w.T, eg_b, br_e)
    eall = _make_edge_pass(n_nodes, E)(ns, nd, ey, src, dst)
    sums = _make_scatter_pass(n_nodes, E, n_chunks)(eall, dst)
    out_pre, nstat = _outpre(sums, su, br_n)
    estat = _stats(eall, br_e)
    out = _apply(out_pre, x, nstat, bnn_g, bnn_b, n_nodes, br_n)
    m2 = _apply(eall, edge_attr, estat, bne_g, bne_b, E, br_e)
    return out, m2


def kernel(x, y, z, edge_index, lg_edge_index,
           n_sg_w, n_sg_b, n_eg_w, n_eg_b, n_su_w, n_su_b, n_du_w, n_du_b,
           n_bnn_g, n_bnn_b, n_bne_g, n_bne_b,
           e_sg_w, e_sg_b, e_eg_w, e_eg_b, e_su_w, e_su_b, e_du_w, e_du_b,
           e_bnn_g, e_bnn_b, e_bne_g, e_bne_b):
    src1, dst1 = edge_index[0], edge_index[1]
    src2, dst2 = lg_edge_index[0], lg_edge_index[1]
    pn = (n_sg_w, n_sg_b, n_eg_w, n_eg_b, n_su_w, n_su_b, n_du_w, n_du_b,
          n_bnn_g, n_bnn_b, n_bne_g, n_bne_b)
    pe = (e_sg_w, e_sg_b, e_eg_w, e_eg_b, e_su_w, e_su_b, e_du_w, e_du_b,
          e_bnn_g, e_bnn_b, e_bne_g, e_bne_b)
    x1, m2 = _egc_layer(x, y, src1, dst1, pn, N_N, 1, br_n=2000, br_e=2000)
    y1, z1 = _egc_layer(m2, z, src2, dst2, pe, N_E, 2, br_n=2000, br_e=2000)
    return (x1, y1, z1)


# speedup vs baseline: 2.4320x; 1.7468x over previous
import functools

import jax
import jax.numpy as jnp
from jax import lax
from jax.experimental import pallas as pl
from jax.experimental.pallas import tpu as pltpu
from jax.experimental.pallas import tpu_sc as plsc

N_N = 10000
N_E = 160000
D = 128
NG = 8
LANES = 16
NW = 32


def _mm3_body(a_ref, w_ref, b_ref, o1_ref, o2_ref, o3_ref):
    r = jnp.dot(a_ref[...], w_ref[...], preferred_element_type=jnp.float32)
    r = r + b_ref[0:1, :]
    o1_ref[...] = r[:, 0 * D:1 * D]
    o2_ref[...] = r[:, 1 * D:2 * D]
    o3_ref[...] = r[:, 2 * D:3 * D]


def _mm3(a, w, b, br):
    R = a.shape[0]
    bt = jnp.broadcast_to(b.reshape(1, 3 * D), (8, 3 * D))
    return pl.pallas_call(
        _mm3_body,
        grid=(R // br,),
        in_specs=[pl.BlockSpec((br, D), lambda i: (i, 0)),
                  pl.BlockSpec((D, 3 * D), lambda i: (0, 0)),
                  pl.BlockSpec((8, 3 * D), lambda i: (0, 0))],
        out_specs=[pl.BlockSpec((br, D), lambda i: (i, 0))] * 3,
        out_shape=[jax.ShapeDtypeStruct((R, D), jnp.float32)] * 3,
    )(a, w, bt)


def _mm1_body(a_ref, w_ref, b_ref, o_ref):
    r = jnp.dot(a_ref[...], w_ref[...], preferred_element_type=jnp.float32)
    o_ref[...] = r + b_ref[0:1, :]


def _mm1(a, w, b, br):
    R = a.shape[0]
    bt = jnp.broadcast_to(b.reshape(1, D), (8, D))
    return pl.pallas_call(
        _mm1_body,
        grid=(R // br,),
        in_specs=[pl.BlockSpec((br, D), lambda i: (i, 0)),
                  pl.BlockSpec((D, D), lambda i: (0, 0)),
                  pl.BlockSpec((8, D), lambda i: (0, 0))],
        out_specs=pl.BlockSpec((br, D), lambda i: (i, 0)),
        out_shape=jax.ShapeDtypeStruct((R, D), jnp.float32),
    )(a, w, bt)


def _outpre_body(sums_ref, su_ref, o_ref, acc_ref):
    i = pl.program_id(0)
    o = sums_ref[1] / (sums_ref[0] + 1e-6) + su_ref[...]
    o_ref[...] = o
    br = o.shape[0]
    ps = o.reshape(br // 8, 8, D).sum(axis=0)
    psq = (o * o).reshape(br // 8, 8, D).sum(axis=0)

    @pl.when(i == 0)
    def _():
        acc_ref[...] = jnp.zeros_like(acc_ref)

    acc_ref[0] = acc_ref[0] + ps
    acc_ref[1] = acc_ref[1] + psq


def _outpre(sums, su, br):
    R = su.shape[0]
    return pl.pallas_call(
        _outpre_body,
        grid=(R // br,),
        in_specs=[pl.BlockSpec((2, br, D), lambda i: (0, i, 0)),
                  pl.BlockSpec((br, D), lambda i: (i, 0))],
        out_specs=[pl.BlockSpec((br, D), lambda i: (i, 0)),
                   pl.BlockSpec((2, 8, D), lambda i: (0, 0, 0))],
        out_shape=[jax.ShapeDtypeStruct((R, D), jnp.float32),
                   jax.ShapeDtypeStruct((2, 8, D), jnp.float32)],
    )(sums, su)


def _stats_body(x_ref, acc_ref):
    i = pl.program_id(0)
    o = x_ref[...]
    br = o.shape[0]
    ps = o.reshape(br // 8, 8, D).sum(axis=0)
    psq = (o * o).reshape(br // 8, 8, D).sum(axis=0)

    @pl.when(i == 0)
    def _():
        acc_ref[...] = jnp.zeros_like(acc_ref)

    acc_ref[0] = acc_ref[0] + ps
    acc_ref[1] = acc_ref[1] + psq


def _stats(x, br):
    R = x.shape[0]
    return pl.pallas_call(
        _stats_body,
        grid=(R // br,),
        in_specs=[pl.BlockSpec((br, D), lambda i: (i, 0))],
        out_specs=pl.BlockSpec((2, 8, D), lambda i: (0, 0, 0)),
        out_shape=jax.ShapeDtypeStruct((2, 8, D), jnp.float32),
    )(x)


def _apply_body(src_ref, res_ref, acc_ref, gb_ref, o_ref, *, count):
    mean = acc_ref[0].sum(axis=0, keepdims=True) * (1.0 / count)
    ex2 = acc_ref[1].sum(axis=0, keepdims=True) * (1.0 / count)
    var = ex2 - mean * mean
    scale = gb_ref[0, 0:1, :] / jnp.sqrt(var + 1e-5)
    xh = (src_ref[...] - mean) * scale + gb_ref[1, 0:1, :]
    o_ref[...] = xh / (1.0 + jnp.exp(-xh)) + res_ref[...]


def _apply(src, res, acc, g, b, count, br):
    R = src.shape[0]
    gb = jnp.stack([jnp.broadcast_to(g.reshape(1, D), (8, D)),
                    jnp.broadcast_to(b.reshape(1, D), (8, D))])
    return pl.pallas_call(
        functools.partial(_apply_body, count=float(count)),
        grid=(R // br,),
        in_specs=[pl.BlockSpec((br, D), lambda i: (i, 0)),
                  pl.BlockSpec((br, D), lambda i: (i, 0)),
                  pl.BlockSpec((2, 8, D), lambda i: (0, 0, 0)),
                  pl.BlockSpec((2, 8, D), lambda i: (0, 0, 0))],
        out_specs=pl.BlockSpec((br, D), lambda i: (i, 0)),
        out_shape=jax.ShapeDtypeStruct((R, D), jnp.float32),
    )(src, res, acc, gb)


@functools.lru_cache(maxsize=None)
def _make_edge_pass(N, E):
    EP = E // NW
    K = 64
    F = EP // K
    TAIL = EP - F * K
    EPP = (F + 1) * K
    assert TAIL and TAIL % 8 == 0 and F >= 4
    P = (F - 2) // 2
    mesh = plsc.VectorSubcoreMesh(core_axis_name="c", subcore_axis_name="s")

    @functools.partial(
        pl.kernel, mesh=mesh,
        compiler_params=pltpu.CompilerParams(use_tc_tiling_on_sc=False),
        out_type=[jax.ShapeDtypeStruct((E, D), jnp.float32)] * 3,
        scratch_types=[pltpu.VMEM((EPP,), jnp.int32),
                       pltpu.VMEM((EPP,), jnp.int32),
                       pltpu.VMEM((2, K, D), jnp.float32),
                       pltpu.VMEM((2, K, D), jnp.float32),
                       pltpu.VMEM((2, K, D), jnp.float32),
                       pltpu.VMEM((2, K, D), jnp.float32),
                       pltpu.VMEM((2, K, D), jnp.float32),
                       pltpu.VMEM((2, K, D), jnp.float32),
                       pltpu.VMEM((2, K, D), jnp.float32),
                       pltpu.SemaphoreType.DMA,
                       pltpu.SemaphoreType.DMA,
                       pltpu.SemaphoreType.DMA,
                       pltpu.SemaphoreType.DMA],
    )
    def edge_pass(ns, du, ey, src, dst, m_out, sg_out, ms_out,
                  sall, dall, av, bv, cv, dv, mv, sv, wv,
                  si0, si1, so0, so1):
        wid = lax.axis_index("s") * 2 + lax.axis_index("c")
        base = wid * EP
        si = (si0, si1)
        so = (so0, so1)
        lane = lax.iota(jnp.int32, LANES)

        bas8 = pl.multiple_of(base, 8)
        pltpu.sync_copy(src.at[pl.ds(bas8, EP)], sall.at[pl.ds(0, EP)])
        pltpu.sync_copy(dst.at[pl.ds(bas8, EP)], dall.at[pl.ds(0, EP)])
        pv = EP // LANES
        rem = EP - pv * LANES
        if rem:
            sl = pl.ds(pv * LANES, LANES)
            keep = lane < rem
            sall[sl] = jnp.where(keep, sall[sl], 0)
            dall[sl] = jnp.where(keep, dall[sl], 0)
        zero = jnp.zeros((LANES,), jnp.int32)
        for j in range(pv + (1 if rem else 0), EPP // LANES):
            sall[pl.ds(j * LANES, LANES)] = zero
            dall[pl.ds(j * LANES, LANES)] = zero

        def in_copies(i, b, k_ey):
            eb = pl.multiple_of(base + i * K, 8)
            ebl = pl.multiple_of(i * K, 8)
            return [(ns.at[dall.at[pl.ds(ebl, K)]], av.at[b]),
                    (ns.at[sall.at[pl.ds(ebl, K)]], bv.at[b]),
                    (du.at[sall.at[pl.ds(ebl, K)]], cv.at[b]),
                    (ey.at[pl.ds(eb, k_ey), :], dv.at[b, pl.ds(0, k_ey), :])]

        def issue_in(i, b, k_ey=K):
            for s, d_ in in_copies(i, b, k_ey):
                pltpu.async_copy(s, d_, si[b])

        def wait_in(i, b, k_ey=K):
            for s, d_ in in_copies(i, b, k_ey):
                pltpu.make_async_copy(s, d_, si[b]).wait()

        def compute(b):
            def row(r, carry):
                for g in range(NG):
                    sl = pl.ds(g * LANES, LANES)
                    mvec = av[b, r, sl] + bv[b, r, sl] + dv[b, r, sl]
                    sg = 1.0 / (1.0 + jnp.exp(-mvec))
                    mv[b, r, sl] = mvec
                    sv[b, r, sl] = sg
                    wv[b, r, sl] = sg * cv[b, r, sl]
                return carry

            lax.fori_loop(0, K, row, 0)

        def out_copies(i, b, k):
            eb = pl.multiple_of(base + i * K, 8)
            return [(mv.at[b, pl.ds(0, k), :], m_out.at[pl.ds(eb, k), :]),
                    (sv.at[b, pl.ds(0, k), :], sg_out.at[pl.ds(eb, k), :]),
                    (wv.at[b, pl.ds(0, k), :], ms_out.at[pl.ds(eb, k), :])]

        def issue_out(i, b, k=K):
            for s, d_ in out_copies(i, b, k):
                pltpu.async_copy(s, d_, so[b])

        def wait_out(b, k=K):
            for s, d_ in out_copies(0, b, k):
                pltpu.make_async_copy(s, d_, so[b]).wait()

        issue_in(F, 0, TAIL)
        wait_in(F, 0, TAIL)
        compute(0)
        issue_out(F, 0, TAIL)
        wait_out(0, TAIL)

        issue_in(0, 0)
        issue_in(1, 1)

        def pair(j, carry):
            i0 = j * 2
            wait_in(i0, 0)
            compute(0)

            @pl.when(j >= 1)
            def _():
                wait_out(0)

            issue_out(i0, 0)
            issue_in(i0 + 2, 0)
            i1 = i0 + 1
            wait_in(i1, 1)
            compute(1)

            @pl.when(j >= 1)
            def _():
                wait_out(1)

            issue_out(i1, 1)
            issue_in(i1 + 2, 1)
            return carry

        lax.fori_loop(0, P, pair, 0)
        for i in range(2 * P + 2, F):
            issue_in(i, i & 1)
        for i in range(2 * P, F):
            b = i & 1
            wait_in(i, b)
            compute(b)
            wait_out(b)
            issue_out(i, b)
        wait_out(0)
        wait_out(1)

    return edge_pass


@functools.lru_cache(maxsize=None)
def _make_scatter_pass(N, E, n_chunks):
    NT = 16
    EP = E // NT
    K = 128
    F = EP // K
    TAIL = EP - F * K
    NROW = F + 1
    SLK = 3
    SL = SLK * K
    NS = F // SLK
    Nc = N // n_chunks
    DUMP = 512
    AR = Nc + DUMP
    RPT = AR // NT
    DRP = Nc // NT
    ZR = 256
    assert TAIL == 16 and F % SLK == 0 and NS % 2 == 0
    assert RPT * NT == AR and DRP * NT == Nc
    mesh = plsc.VectorSubcoreMesh(core_axis_name="c", subcore_axis_name="s")

    @functools.partial(
        pl.kernel, mesh=mesh,
        compiler_params=pltpu.CompilerParams(use_tc_tiling_on_sc=False),
        out_type=jax.ShapeDtypeStruct((2, N, D), jnp.float32),
        scratch_types=[pltpu.VMEM((EP,), jnp.int32),
                       pltpu.VMEM((NROW, K), jnp.int32),
                       pltpu.VMEM((2, SL, LANES), jnp.float32),
                       pltpu.VMEM((ZR, LANES), jnp.float32),
                       pltpu.VMEM_SHARED((AR, LANES), jnp.float32),
                       pltpu.SemaphoreType.DMA,
                       pltpu.SemaphoreType.DMA],
    )
    def scatter_pass(sg_a, ms_a, dst, sums,
                     dstv_all, idxm, slab, zb, acc, sb0, sb1):
        cid = lax.axis_index("c")
        sid = lax.axis_index("s")
        tb = sid * EP
        sb = (sb0, sb1)
        lane = lax.iota(jnp.int32, LANES)

        def zrow(r, carry):
            zb[r, :] = jnp.zeros((LANES,), jnp.float32)
            return carry

        lax.fori_loop(0, ZR, zrow, 0)
        pltpu.sync_copy(dst.at[pl.ds(pl.multiple_of(tb, 8), EP)], dstv_all)

        def masked_idx(dvec, base_n):
            if n_chunks == 1:
                return dvec
            inb = (dvec >= base_n) & (dvec < base_n + Nc)
            return jnp.where(inb, dvec - base_n, Nc + (dvec & (DUMP - 1)))

        def build_idx(c):
            base_n = c * Nc

            def irow(r, carry):
                for j in range(K // LANES):
                    dvec = dstv_all[pl.ds(r * K + j * LANES, LANES)]
                    idxm[r, pl.ds(j * LANES, LANES)] = masked_idx(dvec, base_n)
                return carry

            lax.fori_loop(0, F, irow, 0)
            nv_pad = (K - TAIL) // LANES
            for j in range(K // LANES):
                sl = pl.ds(j * LANES, LANES)
                if j < nv_pad:
                    idxm[F, sl] = Nc + ((lane + j * LANES) & (DUMP - 1))
                else:
                    dvec = dstv_all[pl.ds(EP - K + j * LANES, LANES)]
                    idxm[F, sl] = masked_idx(dvec, base_n)

        def zero_acc():
            r0 = sid * RPT
            nfull, rem = divmod(RPT, ZR)

            def zc(jj, carry):
                pltpu.sync_copy(zb, acc.at[pl.ds(r0 + jj * ZR, ZR)])
                return carry

            lax.fori_loop(0, nfull, zc, 0)
            if rem:
                pltpu.sync_copy(zb.at[pl.ds(0, rem)],
                                acc.at[pl.ds(r0 + nfull * ZR, rem)])

        def accum(t, g):
            tbl = sg_a if t == 0 else ms_a
            co = g * LANES

            def slab_copy(i, b):
                return (tbl.at[pl.ds(pl.multiple_of(tb + i * SL, 8), SL),
                               pl.ds(co, LANES)],
                        slab.at[b, pl.ds(0, SL), :])

            def issue(i, b):
                s_, d_ = slab_copy(i, b)
                pltpu.async_copy(s_, d_, sb[b])

            def wait(i, b):
                s_, d_ = slab_copy(i, b)
                pltpu.make_async_copy(s_, d_, sb[b]).wait()

            def scatters(i, b):
                def sc(jj, carry):
                    pltpu.sync_copy(slab.at[b, pl.ds(jj * K, K), :],
                                    acc.at[idxm.at[i * SLK + jj]], add=True)
                    return carry

                lax.fori_loop(0, SLK, sc, 0)

            issue(0, 0)
            issue(1, 1)

            def spair(j, carry):
                i0 = 2 * j
                wait(i0, 0)
                scatters(i0, 0)

                @pl.when(i0 + 2 < NS)
                def _():
                    issue(i0 + 2, 0)

                wait(i0 + 1, 1)
                scatters(i0 + 1, 1)

                @pl.when(i0 + 3 < NS)
                def _():
                    issue(i0 + 3, 1)

                return carry

            lax.fori_loop(0, NS // 2, spair, 0)
            ts = tbl.at[pl.ds(pl.multiple_of(tb + EP - K, 8), K),
                        pl.ds(co, LANES)]
            td = slab.at[0, pl.ds(0, K), :]
            pltpu.async_copy(ts, td, sb[0])
            pltpu.make_async_copy(ts, td, sb[0]).wait()
            pltpu.sync_copy(slab.at[0, pl.ds(0, K), :],
                            acc.at[idxm.at[F]], add=True)

        def drain(t, c, g):
            n0 = c * Nc + sid * DRP
            pltpu.sync_copy(acc.at[pl.ds(sid * DRP, DRP)],
                            sums.at[t, pl.ds(n0, DRP),
                                    pl.ds(g * LANES, LANES)])

        def run(t):
            for c in range(n_chunks):
                build_idx(c)
                for g in range(NG):
                    zero_acc()
                    plsc.subcore_barrier()
                    accum(t, g)
                    plsc.subcore_barrier()
                    drain(t, c, g)
                    plsc.subcore_barrier()

        @pl.when(cid == 0)
        def _():
            run(0)

        @pl.when(cid == 1)
        def _():
            run(1)

    return scatter_pass


def _egc_layer(x, edge_attr, src, dst, p, n_nodes, n_chunks, br_n, br_e):
    (sg_w, sg_b, eg_w, eg_b, su_w, su_b, du_w, du_b,
     bnn_g, bnn_b, bne_g, bne_b) = p
    E = edge_attr.shape[0]
    w3 = jnp.concatenate([sg_w.T, du_w.T, su_w.T], axis=1)
    b3 = jnp.concatenate([sg_b, du_b, su_b])
    ns, du, su = _mm3(x, w3, b3, br_n)
    ey = _mm1(edge_attr, eg_w.T, eg_b, br_e)
    m, sg_a, ms_a = _make_edge_pass(n_nodes, E)(ns, du, ey, src, dst)
    sums = _make_scatter_pass(n_nodes, E, n_chunks)(sg_a, ms_a, dst)
    out_pre, nstat = _outpre(sums, su, br_n)
    estat = _stats(m, br_e)
    out = _apply(out_pre, x, nstat, bnn_g, bnn_b, n_nodes, br_n)
    m2 = _apply(m, edge_attr, estat, bne_g, bne_b, E, br_e)
    return out, m2


def kernel(x, y, z, edge_index, lg_edge_index,
           n_sg_w, n_sg_b, n_eg_w, n_eg_b, n_su_w, n_su_b, n_du_w, n_du_b,
           n_bnn_g, n_bnn_b, n_bne_g, n_bne_b,
           e_sg_w, e_sg_b, e_eg_w, e_eg_b, e_su_w, e_su_b, e_du_w, e_du_b,
           e_bnn_g, e_bnn_b, e_bne_g, e_bne_b):
    src1, dst1 = edge_index[0], edge_index[1]
    src2, dst2 = lg_edge_index[0], lg_edge_index[1]
    pn = (n_sg_w, n_sg_b, n_eg_w, n_eg_b, n_su_w, n_su_b, n_du_w, n_du_b,
          n_bnn_g, n_bnn_b, n_bne_g, n_bne_b)
    pe = (e_sg_w, e_sg_b, e_eg_w, e_eg_b, e_su_w, e_su_b, e_du_w, e_du_b,
          e_bnn_g, e_bnn_b, e_bne_g, e_bne_b)
    x1, m2 = _egc_layer(x, y, src1, dst1, pn, N_N, 1, br_n=2000, br_e=2000)
    y1, z1 = _egc_layer(m2, z, src2, dst2, pe, N_E, 2, br_n=2000, br_e=2000)
    return (x1, y1, z1)
